# Initial kernel scaffold; baseline (speedup 1.0000x reference)
import jax
import jax.numpy as jnp
from jax.experimental import pallas as pl


def kernel(x, edge_index, W1, att_src1, att_dst1, b1, W2, att_src2, att_dst2, b2, W3, att_src3, att_dst3, b3):
    raise NotImplementedError("write your pallas kernel here")


# speedup vs baseline: 41.1073x; 41.1073x over previous
import functools

import jax
import jax.numpy as jnp
from jax import lax
from jax.experimental import pallas as pl
from jax.experimental.pallas import tpu as pltpu
from jax.experimental.pallas import tpu_sc as plsc

NC = 2
NS = 16
LANES = 16


def _pre_body(x_ref, w_ref, as_ref, ad_ref, h_ref, ab_ref):
    h = jnp.dot(x_ref[...], w_ref[...], preferred_element_type=jnp.float32)
    h_ref[...] = h
    a_src = jnp.dot(h, as_ref[...], preferred_element_type=jnp.float32)
    a_dst = jnp.dot(h, ad_ref[...], preferred_element_type=jnp.float32)
    ab_ref[...] = jnp.concatenate([a_src, a_dst], axis=1)


def _tc_pre(x, w, a_s, a_d):
    n = x.shape[0]
    tw = w.shape[1]
    return pl.pallas_call(
        _pre_body,
        out_shape=[
            jax.ShapeDtypeStruct((n, tw), jnp.float32),
            jax.ShapeDtypeStruct((n, 16), jnp.float32),
        ],
    )(x, w, a_s, a_d)


def _mid_body(heads, p_ref, d_ref, b_ref, exp_ref, w_ref, as_ref, ad_ref,
              h_ref, ab_ref):
    psum = p_ref[0] + p_ref[1]
    den = d_ref[0, :, :heads] + d_ref[1, :, :heads]
    den_w = jnp.dot(den, exp_ref[...], preferred_element_type=jnp.float32)
    g = psum / (den_w + 1e-16) + b_ref[...][None, :]
    g = jnp.where(g > 0, g, jnp.exp(g) - 1.0)
    h = jnp.dot(g, w_ref[...], preferred_element_type=jnp.float32)
    h_ref[...] = h
    a_src = jnp.dot(h, as_ref[...], preferred_element_type=jnp.float32)
    a_dst = jnp.dot(h, ad_ref[...], preferred_element_type=jnp.float32)
    ab_ref[...] = jnp.concatenate([a_src, a_dst], axis=1)


def _tc_mid(heads, p, d, b, expand, w, a_s, a_d):
    n = p.shape[1]
    tw = w.shape[1]
    return pl.pallas_call(
        functools.partial(_mid_body, heads),
        out_shape=[
            jax.ShapeDtypeStruct((n, tw), jnp.float32),
            jax.ShapeDtypeStruct((n, 16), jnp.float32),
        ],
    )(p, d, b, expand, w, a_s, a_d)


def _post_body(p_ref, d_ref, b_ref, o_ref):
    psum = p_ref[0] + p_ref[1]
    den = d_ref[0, :, 0:1] + d_ref[1, :, 0:1]
    z = psum / (den + 1e-16) + b_ref[...][None, :]
    m = jnp.max(z, axis=1, keepdims=True)
    lse = m + jnp.log(jnp.sum(jnp.exp(z - m), axis=1, keepdims=True))
    o_ref[...] = z - lse


def _tc_post(p, d, b):
    n = p.shape[1]
    out = p.shape[2]
    return pl.pallas_call(
        _post_body,
        out_shape=jax.ShapeDtypeStruct((n, out), jnp.float32),
    )(p, d, b)


KB = 80
ZR = 200


def _lane_splat(x, lane):
    idx = jnp.full((LANES,), lane, jnp.int32)
    return lax.gather(
        x, idx[:, None],
        lax.GatherDimensionNumbers(
            offset_dims=(), collapsed_slice_dims=(0,), start_index_map=(0,)),
        (1,), mode=lax.GatherScatterMode.PROMISE_IN_BOUNDS)


def _edge_body(heads, ch, n, e, h_hbm, ab_hbm, src_hbm, dst_hbm, zb_hbm,
               zs_hbm, out_hbm, den_hbm,
               sidx, didx, absrc, abdst, hrows, exb,
               acc, den, sem):
    tw = heads * ch
    c = lax.axis_index("c")
    s = lax.axis_index("s")
    w = s * NC + c
    epw = e // (NC * NS)
    nch = n // ZR
    rounds = (nch + NS - 1) // NS

    for t in range(rounds):
        ci = t * NS + s

        @pl.when(ci < nch)
        def _():
            r0 = pl.multiple_of(ci * ZR, 8)
            pltpu.sync_copy(zb_hbm, acc.at[pl.ds(r0, ZR)])
            pltpu.sync_copy(zs_hbm, den.at[pl.ds(r0, ZR)])

    plsc.subcore_barrier()

    perm = (lax.iota(jnp.int32, LANES) + 8) & 15
    nchunk = tw // LANES

    def blk(b, _):
        off = w * epw + b * KB
        pltpu.sync_copy(src_hbm.at[pl.ds(off, KB)], sidx)
        pltpu.sync_copy(dst_hbm.at[pl.ds(off, KB)], didx)
        pltpu.async_copy(ab_hbm.at[sidx], absrc, sem).wait()
        pltpu.async_copy(ab_hbm.at[didx], abdst, sem).wait()
        pltpu.async_copy(h_hbm.at[sidx], hrows, sem).wait()

        def per_edge(i, _):
            a1 = absrc[i, :]
            a2 = abdst[i, :]
            ev = a1 + lax.gather(
                a2, perm[:, None],
                lax.GatherDimensionNumbers(
                    offset_dims=(), collapsed_slice_dims=(0,),
                    start_index_map=(0,)),
                (1,), mode=lax.GatherScatterMode.PROMISE_IN_BOUNDS)
            ev = jnp.where(ev > 0, ev, 0.2 * ev)
            ex = jnp.exp(ev)
            exb[i, :] = ex
            for j in range(nchunk):
                sp = _lane_splat(ex, j // (ch // LANES))
                hrows[i, pl.ds(LANES * j, LANES)] = (
                    hrows[i, pl.ds(LANES * j, LANES)] * sp)
            return 0

        lax.fori_loop(0, KB, per_edge, 0)
        pltpu.sync_copy(hrows, acc.at[didx], add=True)
        pltpu.sync_copy(exb, den.at[didx], add=True)
        return 0

    lax.fori_loop(0, epw // KB, blk, 0)
    plsc.subcore_barrier()

    for t in range(rounds):
        ci = t * NS + s

        @pl.when(ci < nch)
        def _():
            r0 = pl.multiple_of(ci * ZR, 8)
            pltpu.sync_copy(acc.at[pl.ds(r0, ZR)],
                            out_hbm.at[c, pl.ds(r0, ZR)])
            pltpu.sync_copy(den.at[pl.ds(r0, ZR)],
                            den_hbm.at[c, pl.ds(r0, ZR)])


def _sc_edge_pass(h_tbl, ab_tbl, src, dst, heads, ch):
    n = h_tbl.shape[0]
    e = src.shape[0]
    tw = heads * ch
    mesh = plsc.VectorSubcoreMesh(
        core_axis_name="c", subcore_axis_name="s",
        num_cores=NC, num_subcores=NS)
    kern = pl.kernel(
        functools.partial(_edge_body, heads, ch, n, e),
        out_type=[
            jax.ShapeDtypeStruct((NC, n, tw), jnp.float32),
            jax.ShapeDtypeStruct((NC, n, 16), jnp.float32),
        ],
        mesh=mesh,
        scratch_types=[
            pltpu.VMEM((KB,), jnp.int32),
            pltpu.VMEM((KB,), jnp.int32),
            pltpu.VMEM((KB, 16), jnp.float32),
            pltpu.VMEM((KB, 16), jnp.float32),
            pltpu.VMEM((KB, tw), jnp.float32),
            pltpu.VMEM((KB, 16), jnp.float32),
            pltpu.VMEM_SHARED((n, tw), jnp.float32),
            pltpu.VMEM_SHARED((n, 16), jnp.float32),
            pltpu.SemaphoreType.DMA,
        ],
        compiler_params=pltpu.CompilerParams(use_tc_tiling_on_sc=False),
    )
    zb = jnp.zeros((ZR, tw), jnp.float32)
    zs = jnp.zeros((ZR, 16), jnp.float32)
    return kern(h_tbl, ab_tbl, src, dst, zb, zs)


def _att_mat(att, heads, ch):
    a = att.reshape(heads, ch)
    m = jnp.eye(heads, dtype=jnp.float32)[:, None, :] * a[:, :, None]
    m = m.reshape(heads * ch, heads)
    if heads < 8:
        m = jnp.concatenate(
            [m, jnp.zeros((heads * ch, 8 - heads), jnp.float32)], axis=1)
    return m


def _expand_mat(heads, ch):
    m = jnp.eye(heads, dtype=jnp.float32)[:, :, None] * jnp.ones(
        (1, 1, ch), jnp.float32)
    return m.reshape(heads, heads * ch)


def kernel(x, edge_index, W1, att_src1, att_dst1, b1, W2, att_src2, att_dst2,
           b2, W3, att_src3, att_dst3, b3):
    src = edge_index[0]
    dst = edge_index[1]
    h1, ab1 = _tc_pre(x, W1, _att_mat(att_src1, 8, 16),
                      _att_mat(att_dst1, 8, 16))
    p1, d1 = _sc_edge_pass(h1, ab1, src, dst, 8, 16)

    h2, ab2 = _tc_mid(8, p1, d1, b1, _expand_mat(8, 16), W2,
                      _att_mat(att_src2, 8, 16), _att_mat(att_dst2, 8, 16))
    p2, d2 = _sc_edge_pass(h2, ab2, src, dst, 8, 16)

    h3, ab3 = _tc_mid(8, p2, d2, b2, _expand_mat(8, 16), W3,
                      _att_mat(att_src3, 1, 64), _att_mat(att_dst3, 1, 64))
    p3, d3 = _sc_edge_pass(h3, ab3, src, dst, 1, 64)

    return _tc_post(p3, d3, b3)


# speedup vs baseline: 83.3364x; 2.0273x over previous
import functools

import jax
import jax.numpy as jnp
from jax import lax
from jax.experimental import pallas as pl
from jax.experimental.pallas import tpu as pltpu
from jax.experimental.pallas import tpu_sc as plsc

NC = 2
NS = 16
LANES = 16


def _pre_body(x_ref, w_ref, as_ref, ad_ref, h_ref, ab_ref):
    h = jnp.dot(x_ref[...], w_ref[...], preferred_element_type=jnp.float32)
    h_ref[...] = h
    a_src = jnp.dot(h, as_ref[...], preferred_element_type=jnp.float32)
    a_dst = jnp.dot(h, ad_ref[...], preferred_element_type=jnp.float32)
    ab_ref[...] = jnp.concatenate([a_src, a_dst], axis=1)


def _tc_pre(x, w, a_s, a_d):
    n = x.shape[0]
    tw = w.shape[1]
    return pl.pallas_call(
        _pre_body,
        out_shape=[
            jax.ShapeDtypeStruct((n, tw), jnp.float32),
            jax.ShapeDtypeStruct((n, 16), jnp.float32),
        ],
    )(x, w, a_s, a_d)


def _mid_body(heads, p_ref, d_ref, b_ref, exp_ref, w_ref, as_ref, ad_ref,
              h_ref, ab_ref):
    psum = p_ref[0] + p_ref[1]
    den = d_ref[0, :, :heads] + d_ref[1, :, :heads]
    den_w = jnp.dot(den, exp_ref[...], preferred_element_type=jnp.float32)
    g = psum / (den_w + 1e-16) + b_ref[...][None, :]
    g = jnp.where(g > 0, g, jnp.exp(g) - 1.0)
    h = jnp.dot(g, w_ref[...], preferred_element_type=jnp.float32)
    h_ref[...] = h
    a_src = jnp.dot(h, as_ref[...], preferred_element_type=jnp.float32)
    a_dst = jnp.dot(h, ad_ref[...], preferred_element_type=jnp.float32)
    ab_ref[...] = jnp.concatenate([a_src, a_dst], axis=1)


def _tc_mid(heads, p, d, b, expand, w, a_s, a_d):
    n = p.shape[1]
    tw = w.shape[1]
    return pl.pallas_call(
        functools.partial(_mid_body, heads),
        out_shape=[
            jax.ShapeDtypeStruct((n, tw), jnp.float32),
            jax.ShapeDtypeStruct((n, 16), jnp.float32),
        ],
    )(p, d, b, expand, w, a_s, a_d)


def _post_body(p_ref, d_ref, b_ref, o_ref):
    psum = p_ref[0] + p_ref[1]
    den = d_ref[0, :, 0:1] + d_ref[1, :, 0:1]
    z = psum / (den + 1e-16) + b_ref[...][None, :]
    m = jnp.max(z, axis=1, keepdims=True)
    lse = m + jnp.log(jnp.sum(jnp.exp(z - m), axis=1, keepdims=True))
    o_ref[...] = z - lse


def _tc_post(p, d, b):
    n = p.shape[1]
    out = p.shape[2]
    return pl.pallas_call(
        _post_body,
        out_shape=jax.ShapeDtypeStruct((n, out), jnp.float32),
    )(p, d, b)


KB = 80
ZR = 200


def _lane_splat(x, lane):
    idx = jnp.full((LANES,), lane, jnp.int32)
    return lax.gather(
        x, idx[:, None],
        lax.GatherDimensionNumbers(
            offset_dims=(), collapsed_slice_dims=(0,), start_index_map=(0,)),
        (1,), mode=lax.GatherScatterMode.PROMISE_IN_BOUNDS)


CROWS = 25


def _edge_body(heads, ch, n, e, h_hbm, ab_hbm, src_hbm, dst_hbm, zb_hbm,
               zs_hbm, out_hbm, den_hbm,
               sidx, didx, absrc0, abdst0, hrows0, absrc1, abdst1, hrows1,
               acc, den, sem0, sem1):
    tw = heads * ch
    c = lax.axis_index("c")
    s = lax.axis_index("s")
    w = s * NC + c
    nblk = (e // KB) // (NC * NS)
    row0 = w * nblk
    nch = n // ZR
    rounds = (nch + NS - 1) // NS

    for t in range(rounds):
        ci = t * NS + s

        @pl.when(ci < nch)
        def _():
            r0 = pl.multiple_of(ci * ZR, 8)
            pltpu.sync_copy(zb_hbm, acc.at[pl.ds(r0, ZR)])
            pltpu.sync_copy(zs_hbm, den.at[pl.ds(r0, ZR)])

    plsc.subcore_barrier()

    perm = (lax.iota(jnp.int32, LANES) + 8) & 15
    nchunk = tw // LANES
    bufs = ((absrc0, abdst0, hrows0, sem0), (absrc1, abdst1, hrows1, sem1))

    def issue(r, buf):
        ab_s, ab_d, hr, sem = buf
        pltpu.async_copy(ab_hbm.at[sidx.at[r]], ab_s, sem)
        pltpu.async_copy(ab_hbm.at[didx.at[r]], ab_d, sem)
        pltpu.async_copy(h_hbm.at[sidx.at[r]], hr, sem)

    def wait(buf):
        ab_s, ab_d, hr, sem = buf
        pltpu.make_async_copy(ab_hbm.at[pl.ds(0, KB)], ab_s, sem).wait()
        pltpu.make_async_copy(ab_hbm.at[pl.ds(0, KB)], ab_d, sem).wait()
        pltpu.make_async_copy(h_hbm.at[pl.ds(0, KB)], hr, sem).wait()

    def compute_scatter(r, buf):
        ab_s, ab_d, hr, _ = buf

        def per_edge(i, _):
            a1 = ab_s[i, :]
            a2 = ab_d[i, :]
            ev = a1 + lax.gather(
                a2, perm[:, None],
                lax.GatherDimensionNumbers(
                    offset_dims=(), collapsed_slice_dims=(0,),
                    start_index_map=(0,)),
                (1,), mode=lax.GatherScatterMode.PROMISE_IN_BOUNDS)
            ev = jnp.where(ev > 0, ev, 0.2 * ev)
            ex = jnp.exp(ev)
            ab_s[i, :] = ex
            for j in range(nchunk):
                sp = _lane_splat(ex, j // (ch // LANES))
                hr[i, pl.ds(LANES * j, LANES)] = (
                    hr[i, pl.ds(LANES * j, LANES)] * sp)
            return 0

        lax.fori_loop(0, KB, per_edge, 0)
        pltpu.sync_copy(hr, acc.at[didx.at[r]], add=True)
        pltpu.sync_copy(ab_s, den.at[didx.at[r]], add=True)

    for chunk in range(nblk // CROWS):
        crow = row0 + chunk * CROWS
        pltpu.sync_copy(src_hbm.at[pl.ds(crow, CROWS), :], sidx)
        pltpu.sync_copy(dst_hbm.at[pl.ds(crow, CROWS), :], didx)
        issue(0, bufs[0])

        def pair(i, _):
            issue(2 * i + 1, bufs[1])
            wait(bufs[0])
            compute_scatter(2 * i, bufs[0])
            issue(2 * i + 2, bufs[0])
            wait(bufs[1])
            compute_scatter(2 * i + 1, bufs[1])
            return 0

        lax.fori_loop(0, (CROWS - 1) // 2, pair, 0)
        wait(bufs[0])
        compute_scatter(CROWS - 1, bufs[0])

    plsc.subcore_barrier()

    for t in range(rounds):
        ci = t * NS + s

        @pl.when(ci < nch)
        def _():
            r0 = pl.multiple_of(ci * ZR, 8)
            pltpu.sync_copy(acc.at[pl.ds(r0, ZR)],
                            out_hbm.at[c, pl.ds(r0, ZR)])
            pltpu.sync_copy(den.at[pl.ds(r0, ZR)],
                            den_hbm.at[c, pl.ds(r0, ZR)])


def _sc_edge_pass(h_tbl, ab_tbl, src, dst, heads, ch):
    n = h_tbl.shape[0]
    e = src.shape[0]
    tw = heads * ch
    mesh = plsc.VectorSubcoreMesh(
        core_axis_name="c", subcore_axis_name="s",
        num_cores=NC, num_subcores=NS)
    kern = pl.kernel(
        functools.partial(_edge_body, heads, ch, n, e),
        out_type=[
            jax.ShapeDtypeStruct((NC, n, tw), jnp.float32),
            jax.ShapeDtypeStruct((NC, n, 16), jnp.float32),
        ],
        mesh=mesh,
        scratch_types=[
            pltpu.VMEM((CROWS, KB), jnp.int32),
            pltpu.VMEM((CROWS, KB), jnp.int32),
            pltpu.VMEM((KB, 16), jnp.float32),
            pltpu.VMEM((KB, 16), jnp.float32),
            pltpu.VMEM((KB, tw), jnp.float32),
            pltpu.VMEM((KB, 16), jnp.float32),
            pltpu.VMEM((KB, 16), jnp.float32),
            pltpu.VMEM((KB, tw), jnp.float32),
            pltpu.VMEM_SHARED((n, tw), jnp.float32),
            pltpu.VMEM_SHARED((n, 16), jnp.float32),
            pltpu.SemaphoreType.DMA,
            pltpu.SemaphoreType.DMA,
        ],
        compiler_params=pltpu.CompilerParams(use_tc_tiling_on_sc=False),
    )
    zb = jnp.zeros((ZR, tw), jnp.float32)
    zs = jnp.zeros((ZR, 16), jnp.float32)
    return kern(h_tbl, ab_tbl, src.reshape(e // KB, KB),
                dst.reshape(e // KB, KB), zb, zs)


def _att_mat(att, heads, ch):
    a = att.reshape(heads, ch)
    m = jnp.eye(heads, dtype=jnp.float32)[:, None, :] * a[:, :, None]
    m = m.reshape(heads * ch, heads)
    if heads < 8:
        m = jnp.concatenate(
            [m, jnp.zeros((heads * ch, 8 - heads), jnp.float32)], axis=1)
    return m


def _expand_mat(heads, ch):
    m = jnp.eye(heads, dtype=jnp.float32)[:, :, None] * jnp.ones(
        (1, 1, ch), jnp.float32)
    return m.reshape(heads, heads * ch)


def kernel(x, edge_index, W1, att_src1, att_dst1, b1, W2, att_src2, att_dst2,
           b2, W3, att_src3, att_dst3, b3):
    src = edge_index[0]
    dst = edge_index[1]
    h1, ab1 = _tc_pre(x, W1, _att_mat(att_src1, 8, 16),
                      _att_mat(att_dst1, 8, 16))
    p1, d1 = _sc_edge_pass(h1, ab1, src, dst, 8, 16)

    h2, ab2 = _tc_mid(8, p1, d1, b1, _expand_mat(8, 16), W2,
                      _att_mat(att_src2, 8, 16), _att_mat(att_dst2, 8, 16))
    p2, d2 = _sc_edge_pass(h2, ab2, src, dst, 8, 16)

    h3, ab3 = _tc_mid(8, p2, d2, b2, _expand_mat(8, 16), W3,
                      _att_mat(att_src3, 1, 64), _att_mat(att_dst3, 1, 64))
    p3, d3 = _sc_edge_pass(h3, ab3, src, dst, 1, 64)

    return _tc_post(p3, d3, b3)


# speedup vs baseline: 142.8034x; 1.7136x over previous
import functools

import jax
import jax.numpy as jnp
from jax import lax
from jax.experimental import pallas as pl
from jax.experimental.pallas import tpu as pltpu
from jax.experimental.pallas import tpu_sc as plsc

NC = 2
NS = 16
LANES = 16


def _pre_body(x_ref, w_ref, as_ref, ad_ref, h_ref, ab_ref):
    h = jnp.dot(x_ref[...], w_ref[...], preferred_element_type=jnp.float32)
    h_ref[...] = h
    a_src = jnp.dot(h, as_ref[...], preferred_element_type=jnp.float32)
    a_dst = jnp.dot(h, ad_ref[...], preferred_element_type=jnp.float32)
    ab_ref[...] = jnp.concatenate([a_src, a_dst], axis=1)


def _tc_pre(x, w, a_s, a_d):
    n = x.shape[0]
    tw = w.shape[1]
    return pl.pallas_call(
        _pre_body,
        out_shape=[
            jax.ShapeDtypeStruct((n, tw), jnp.float32),
            jax.ShapeDtypeStruct((n, 16), jnp.float32),
        ],
    )(x, w, a_s, a_d)


def _mid_body(heads, p_ref, d_ref, b_ref, exp_ref, w_ref, as_ref, ad_ref,
              h_ref, ab_ref):
    psum = p_ref[0] + p_ref[1]
    den = d_ref[0, :, :heads] + d_ref[1, :, :heads]
    den_w = jnp.dot(den, exp_ref[...], preferred_element_type=jnp.float32)
    g = psum / (den_w + 1e-16) + b_ref[...][None, :]
    g = jnp.where(g > 0, g, jnp.exp(g) - 1.0)
    h = jnp.dot(g, w_ref[...], preferred_element_type=jnp.float32)
    h_ref[...] = h
    a_src = jnp.dot(h, as_ref[...], preferred_element_type=jnp.float32)
    a_dst = jnp.dot(h, ad_ref[...], preferred_element_type=jnp.float32)
    ab_ref[...] = jnp.concatenate([a_src, a_dst], axis=1)


def _tc_mid(heads, p, d, b, expand, w, a_s, a_d):
    n = p.shape[1]
    tw = w.shape[1]
    return pl.pallas_call(
        functools.partial(_mid_body, heads),
        out_shape=[
            jax.ShapeDtypeStruct((n, tw), jnp.float32),
            jax.ShapeDtypeStruct((n, 16), jnp.float32),
        ],
    )(p, d, b, expand, w, a_s, a_d)


def _post_body(p_ref, d_ref, b_ref, o_ref):
    psum = p_ref[0] + p_ref[1]
    den = d_ref[0, :, 0:1] + d_ref[1, :, 0:1]
    z = psum / (den + 1e-16) + b_ref[...][None, :]
    m = jnp.max(z, axis=1, keepdims=True)
    lse = m + jnp.log(jnp.sum(jnp.exp(z - m), axis=1, keepdims=True))
    o_ref[...] = z - lse


def _tc_post(p, d, b):
    n = p.shape[1]
    out = p.shape[2]
    return pl.pallas_call(
        _post_body,
        out_shape=jax.ShapeDtypeStruct((n, out), jnp.float32),
    )(p, d, b)


KB = 80
ZR = 200


def _lane_splat(x, lane):
    idx = jnp.full((LANES,), lane, jnp.int32)
    return lax.gather(
        x, idx[:, None],
        lax.GatherDimensionNumbers(
            offset_dims=(), collapsed_slice_dims=(0,), start_index_map=(0,)),
        (1,), mode=lax.GatherScatterMode.PROMISE_IN_BOUNDS)


CROWS = 25


def _edge_body(heads, ch, n, e, h_hbm, ab_hbm, src_hbm, dst_hbm, zb_hbm,
               zs_hbm, out_hbm, den_hbm,
               sidx, didx, absrc0, abdst0, hrows0, absrc1, abdst1, hrows1,
               acc, den, sem0, sem1):
    tw = heads * ch
    c = lax.axis_index("c")
    s = lax.axis_index("s")
    w = s * NC + c
    nblk = (e // KB) // (NC * NS)
    row0 = w * nblk
    nch = n // ZR
    rounds = (nch + NS - 1) // NS

    for t in range(rounds):
        ci = t * NS + s

        @pl.when(ci < nch)
        def _():
            r0 = pl.multiple_of(ci * ZR, 8)
            pltpu.sync_copy(zb_hbm, acc.at[pl.ds(r0, ZR)])
            pltpu.sync_copy(zs_hbm, den.at[pl.ds(r0, ZR)])

    plsc.subcore_barrier()

    perm = (lax.iota(jnp.int32, LANES) + 8) & 15
    nchunk = tw // LANES
    bufs = ((absrc0, abdst0, hrows0, sem0), (absrc1, abdst1, hrows1, sem1))

    def issue(r, buf):
        ab_s, ab_d, hr, sem = buf
        pltpu.async_copy(ab_hbm.at[sidx.at[r]], ab_s, sem)
        pltpu.async_copy(ab_hbm.at[didx.at[r]], ab_d, sem)
        pltpu.async_copy(h_hbm.at[sidx.at[r]], hr, sem)

    def wait(buf):
        ab_s, ab_d, hr, sem = buf
        pltpu.make_async_copy(ab_hbm.at[pl.ds(0, KB)], ab_s, sem).wait()
        pltpu.make_async_copy(ab_hbm.at[pl.ds(0, KB)], ab_d, sem).wait()
        pltpu.make_async_copy(h_hbm.at[pl.ds(0, KB)], hr, sem).wait()

    def compute_scatter(r, buf):
        ab_s, ab_d, hr, _ = buf

        @plsc.parallel_loop(0, KB, unroll=4)
        def per_edge(i):
            a1 = ab_s[i, :]
            a2 = ab_d[i, :]
            ev = a1 + lax.gather(
                a2, perm[:, None],
                lax.GatherDimensionNumbers(
                    offset_dims=(), collapsed_slice_dims=(0,),
                    start_index_map=(0,)),
                (1,), mode=lax.GatherScatterMode.PROMISE_IN_BOUNDS)
            ev = jnp.where(ev > 0, ev, 0.2 * ev)
            ex = jnp.exp(ev)
            ab_s[i, :] = ex
            for j in range(nchunk):
                sp = _lane_splat(ex, j // (ch // LANES))
                hr[i, pl.ds(LANES * j, LANES)] = (
                    hr[i, pl.ds(LANES * j, LANES)] * sp)
        pltpu.sync_copy(hr, acc.at[didx.at[r]], add=True)
        pltpu.sync_copy(ab_s, den.at[didx.at[r]], add=True)

    for chunk in range(nblk // CROWS):
        crow = row0 + chunk * CROWS
        pltpu.sync_copy(src_hbm.at[pl.ds(crow, CROWS), :], sidx)
        pltpu.sync_copy(dst_hbm.at[pl.ds(crow, CROWS), :], didx)
        issue(0, bufs[0])

        def pair(i, _):
            issue(2 * i + 1, bufs[1])
            wait(bufs[0])
            compute_scatter(2 * i, bufs[0])
            issue(2 * i + 2, bufs[0])
            wait(bufs[1])
            compute_scatter(2 * i + 1, bufs[1])
            return 0

        lax.fori_loop(0, (CROWS - 1) // 2, pair, 0)
        wait(bufs[0])
        compute_scatter(CROWS - 1, bufs[0])

    plsc.subcore_barrier()

    for t in range(rounds):
        ci = t * NS + s

        @pl.when(ci < nch)
        def _():
            r0 = pl.multiple_of(ci * ZR, 8)
            pltpu.sync_copy(acc.at[pl.ds(r0, ZR)],
                            out_hbm.at[c, pl.ds(r0, ZR)])
            pltpu.sync_copy(den.at[pl.ds(r0, ZR)],
                            den_hbm.at[c, pl.ds(r0, ZR)])


def _sc_edge_pass(h_tbl, ab_tbl, src, dst, heads, ch):
    n = h_tbl.shape[0]
    e = src.shape[0]
    tw = heads * ch
    mesh = plsc.VectorSubcoreMesh(
        core_axis_name="c", subcore_axis_name="s",
        num_cores=NC, num_subcores=NS)
    kern = pl.kernel(
        functools.partial(_edge_body, heads, ch, n, e),
        out_type=[
            jax.ShapeDtypeStruct((NC, n, tw), jnp.float32),
            jax.ShapeDtypeStruct((NC, n, 16), jnp.float32),
        ],
        mesh=mesh,
        scratch_types=[
            pltpu.VMEM((CROWS, KB), jnp.int32),
            pltpu.VMEM((CROWS, KB), jnp.int32),
            pltpu.VMEM((KB, 16), jnp.float32),
            pltpu.VMEM((KB, 16), jnp.float32),
            pltpu.VMEM((KB, tw), jnp.float32),
            pltpu.VMEM((KB, 16), jnp.float32),
            pltpu.VMEM((KB, 16), jnp.float32),
            pltpu.VMEM((KB, tw), jnp.float32),
            pltpu.VMEM_SHARED((n, tw), jnp.float32),
            pltpu.VMEM_SHARED((n, 16), jnp.float32),
            pltpu.SemaphoreType.DMA,
            pltpu.SemaphoreType.DMA,
        ],
        compiler_params=pltpu.CompilerParams(use_tc_tiling_on_sc=False),
    )
    zb = jnp.zeros((ZR, tw), jnp.float32)
    zs = jnp.zeros((ZR, 16), jnp.float32)
    return kern(h_tbl, ab_tbl, src.reshape(e // KB, KB),
                dst.reshape(e // KB, KB), zb, zs)


def _att_mat(att, heads, ch):
    a = att.reshape(heads, ch)
    m = jnp.eye(heads, dtype=jnp.float32)[:, None, :] * a[:, :, None]
    m = m.reshape(heads * ch, heads)
    if heads < 8:
        m = jnp.concatenate(
            [m, jnp.zeros((heads * ch, 8 - heads), jnp.float32)], axis=1)
    return m


def _expand_mat(heads, ch):
    m = jnp.eye(heads, dtype=jnp.float32)[:, :, None] * jnp.ones(
        (1, 1, ch), jnp.float32)
    return m.reshape(heads, heads * ch)


def kernel(x, edge_index, W1, att_src1, att_dst1, b1, W2, att_src2, att_dst2,
           b2, W3, att_src3, att_dst3, b3):
    src = edge_index[0]
    dst = edge_index[1]
    h1, ab1 = _tc_pre(x, W1, _att_mat(att_src1, 8, 16),
                      _att_mat(att_dst1, 8, 16))
    p1, d1 = _sc_edge_pass(h1, ab1, src, dst, 8, 16)

    h2, ab2 = _tc_mid(8, p1, d1, b1, _expand_mat(8, 16), W2,
                      _att_mat(att_src2, 8, 16), _att_mat(att_dst2, 8, 16))
    p2, d2 = _sc_edge_pass(h2, ab2, src, dst, 8, 16)

    h3, ab3 = _tc_mid(8, p2, d2, b2, _expand_mat(8, 16), W3,
                      _att_mat(att_src3, 1, 64), _att_mat(att_dst3, 1, 64))
    p3, d3 = _sc_edge_pass(h3, ab3, src, dst, 1, 64)

    return _tc_post(p3, d3, b3)
